# Initial kernel scaffold; baseline (speedup 1.0000x reference)
import jax
import jax.numpy as jnp
from jax.experimental import pallas as pl


def kernel(feature_ids, feature_vals, embedding_table):
    raise NotImplementedError("write your pallas kernel here")


# speedup vs baseline: 1.1576x; 1.1576x over previous
import functools

import jax
import jax.numpy as jnp
from jax import lax
from jax.experimental import pallas as pl
from jax.experimental.pallas import tpu as pltpu
from jax.experimental.pallas import tpu_sc as plsc

B = 16384
F = 26
D = 16
NC, NS = 2, 16
NW = NC * NS
ROWS_PW = B // NW
CB = 64
NCHUNK = ROWS_PW // CB
IDX_PC = CB * F
SEG = 128
NSEG = IDX_PC // SEG


def _nfm_body(ids_hbm, vals_hbm, table_hbm, out_hbm,
              ids_v, vals_v, rows_v, out_v, gsem):
    wid = lax.axis_index("s") * NC + lax.axis_index("c")

    def chunk_body(ci, carry):
        base = wid * ROWS_PW + ci * CB
        fbase = base * F
        pltpu.sync_copy(ids_hbm.at[pl.ds(fbase, IDX_PC)], ids_v)
        pltpu.sync_copy(vals_hbm.at[pl.ds(fbase, IDX_PC)],
                        vals_v.at[pl.ds(0, IDX_PC)])
        copies = [
            pltpu.async_copy(
                table_hbm.at[ids_v.at[pl.ds(j * SEG, SEG)]],
                rows_v.at[pl.ds(j * SEG, SEG), :],
                gsem,
            )
            for j in range(NSEG)
        ]
        for c in copies:
            c.wait()

        def row_body(b, carry2):
            bb = b * F
            va = vals_v[pl.ds(bb, 16)]
            vb = vals_v[pl.ds(bb + 16, 16)]
            s = jnp.zeros((16,), jnp.float32)
            q = jnp.zeros((16,), jnp.float32)
            for f in range(F):
                v = jnp.full((16,), va[f] if f < 16 else vb[f - 16],
                             jnp.float32)
                r = rows_v[bb + f, :]
                t = v * r
                s = s + t
                q = q + t * t
            out_v[b, :] = 0.5 * s * s - q
            return carry2

        lax.fori_loop(0, CB, row_body, 0, unroll=2)
        pltpu.sync_copy(out_v, out_hbm.at[pl.ds(base, CB)])
        return carry

    lax.fori_loop(0, NCHUNK, chunk_body, 0)


@jax.jit
def _nfm_sc(ids_flat, vals_flat, table):
    mesh = plsc.VectorSubcoreMesh(core_axis_name="c", subcore_axis_name="s")
    return pl.kernel(
        _nfm_body,
        out_type=jax.ShapeDtypeStruct((B, D), jnp.float32),
        mesh=mesh,
        compiler_params=pltpu.CompilerParams(use_tc_tiling_on_sc=False),
        scratch_types=[
            pltpu.VMEM((IDX_PC,), jnp.int32),
            pltpu.VMEM((IDX_PC + 32,), jnp.float32),
            pltpu.VMEM((IDX_PC, D), jnp.float32),
            pltpu.VMEM((CB, D), jnp.float32),
            pltpu.SemaphoreType.DMA,
        ],
    )(ids_flat, vals_flat, table)


def kernel(feature_ids, feature_vals, embedding_table):
    ids_flat = feature_ids.reshape(-1).astype(jnp.int32)
    vals_flat = feature_vals.reshape(-1)
    return _nfm_sc(ids_flat, vals_flat, embedding_table)


# speedup vs baseline: 1.2128x; 1.0477x over previous
import functools

import jax
import jax.numpy as jnp
from jax import lax
from jax.experimental import pallas as pl
from jax.experimental.pallas import tpu as pltpu
from jax.experimental.pallas import tpu_sc as plsc

B = 16384
F = 26
D = 16
NROWS = 1000000
NC, NS = 2, 16
NW = NC * NS
ROWS_PW = B // NW
CB = 64
NCHUNK = ROWS_PW // CB
IDX_PC = CB * F
SEG = 128
NSEG = IDX_PC // SEG

RBLK = 2048
RGRID = -(-NROWS // RBLK)
PROWS = RGRID * RBLK


def _retile_body(x_ref, o_ref):
    z = x_ref[...].T
    for j in range(8):
        o_ref[:, j * D:(j + 1) * D] = z[j * (RBLK // 8):(j + 1) * (RBLK // 8), :]


@jax.jit
def _retile(table_t):
    return pl.pallas_call(
        _retile_body,
        grid=(RGRID,),
        in_specs=[pl.BlockSpec((D, RBLK), lambda j: (0, j))],
        out_specs=pl.BlockSpec((RBLK * D // 128, 128), lambda j: (j, 0)),
        out_shape=jax.ShapeDtypeStruct((PROWS * D // 128, 128), jnp.float32),
    )(table_t)


def _nfm_body(ids_hbm, vals_hbm, table_hbm, out_hbm,
              ids_v, vals_v, idx_v, rows_v, out_v, gsem):
    wid = lax.axis_index("s") * NC + lax.axis_index("c")

    def chunk_body(ci, carry):
        base = wid * ROWS_PW + ci * CB
        fbase = base * F
        pltpu.sync_copy(ids_hbm.at[pl.ds(fbase, IDX_PC)], ids_v)
        pltpu.sync_copy(vals_hbm.at[pl.ds(fbase, IDX_PC)],
                        vals_v.at[pl.ds(0, IDX_PC)])

        def perm_body(k, carry2):
            i = ids_v[pl.ds(k * 16, 16)]
            row = ((i & ~jnp.int32(2047))
                   + ((i & jnp.int32(255)) << 3)
                   + ((i >> 8) & jnp.int32(7)))
            idx_v[pl.ds(k * 16, 16)] = row
            return carry2

        lax.fori_loop(0, IDX_PC // 16, perm_body, 0, unroll=4)

        copies = [
            pltpu.async_copy(
                table_hbm.at[idx_v.at[pl.ds(j * SEG, SEG)]],
                rows_v.at[pl.ds(j * SEG, SEG), :],
                gsem,
            )
            for j in range(NSEG)
        ]
        for c in copies:
            c.wait()

        def row_body(b, carry2):
            bb = b * F
            va = vals_v[pl.ds(bb, 16)]
            vb = vals_v[pl.ds(bb + 16, 16)]
            s = jnp.zeros((16,), jnp.float32)
            q = jnp.zeros((16,), jnp.float32)
            for f in range(F):
                v = jnp.full((16,), va[f] if f < 16 else vb[f - 16],
                             jnp.float32)
                r = rows_v[bb + f, :]
                t = v * r
                s = s + t
                q = q + t * t
            out_v[b, :] = 0.5 * s * s - q
            return carry2

        lax.fori_loop(0, CB, row_body, 0, unroll=2)
        pltpu.sync_copy(out_v, out_hbm.at[pl.ds(base, CB)])
        return carry

    lax.fori_loop(0, NCHUNK, chunk_body, 0)


@jax.jit
def _nfm_sc(ids_flat, vals_flat, table):
    mesh = plsc.VectorSubcoreMesh(core_axis_name="c", subcore_axis_name="s")
    return pl.kernel(
        _nfm_body,
        out_type=jax.ShapeDtypeStruct((B, D), jnp.float32),
        mesh=mesh,
        compiler_params=pltpu.CompilerParams(use_tc_tiling_on_sc=False),
        scratch_types=[
            pltpu.VMEM((IDX_PC,), jnp.int32),
            pltpu.VMEM((IDX_PC + 32,), jnp.float32),
            pltpu.VMEM((IDX_PC,), jnp.int32),
            pltpu.VMEM((IDX_PC, D), jnp.float32),
            pltpu.VMEM((CB, D), jnp.float32),
            pltpu.SemaphoreType.DMA,
        ],
    )(ids_flat, vals_flat, table)


def kernel(feature_ids, feature_vals, embedding_table):
    ids_flat = feature_ids.reshape(-1).astype(jnp.int32)
    vals_flat = feature_vals.reshape(-1)
    table128 = _retile(embedding_table.T)
    table_lin = table128.reshape(-1).reshape(PROWS, D)
    return _nfm_sc(ids_flat, vals_flat, table_lin)


# speedup vs baseline: 2.8545x; 2.3537x over previous
import functools

import jax
import jax.numpy as jnp
from jax import lax
from jax.experimental import pallas as pl
from jax.experimental.pallas import tpu as pltpu
from jax.experimental.pallas import tpu_sc as plsc

B = 16384
F = 26
D = 16
NROWS = 1000000
NC, NS = 2, 16
NW = NC * NS
ROWS_PW = B // NW
CB = 64
NCHUNK = ROWS_PW // CB
IDX_PC = CB * F
SEG = 128
NSEG = IDX_PC // SEG

RBLK = 8192
RGRID = -(-NROWS // RBLK)
PROWS = RGRID * RBLK


def _retile_body(x_ref, o_ref):
    for m in range(RBLK // 1024):
        X = jnp.concatenate(
            [x_ref[:, (8 * m + k) * 128:(8 * m + k + 1) * 128]
             for k in range(8)], axis=0)
        o_ref[m * 128:(m + 1) * 128, :] = X.T


@jax.jit
def _retile(table_t):
    return pl.pallas_call(
        _retile_body,
        grid=(RGRID,),
        in_specs=[pl.BlockSpec((D, RBLK), lambda j: (0, j))],
        out_specs=pl.BlockSpec((RBLK * D // 128, 128), lambda j: (j, 0)),
        out_shape=jax.ShapeDtypeStruct((PROWS * D // 128, 128), jnp.float32),
    )(table_t)


def _nfm_body(ids_hbm, vals_hbm, table_hbm, out_hbm,
              ids_v, vals_v, idx_v, rows_v, out_v, gsem):
    wid = lax.axis_index("s") * NC + lax.axis_index("c")

    def chunk_body(ci, carry):
        base = wid * ROWS_PW + ci * CB
        fbase = base * F
        pltpu.sync_copy(ids_hbm.at[pl.ds(fbase, IDX_PC)], ids_v)
        pltpu.sync_copy(vals_hbm.at[pl.ds(fbase, IDX_PC)],
                        vals_v.at[pl.ds(0, IDX_PC)])

        def perm_body(k, carry2):
            i = ids_v[pl.ds(k * 16, 16)]
            row = ((i & ~jnp.int32(1023))
                   + ((i & jnp.int32(127)) << 3)
                   + ((i >> 7) & jnp.int32(7)))
            idx_v[pl.ds(k * 16, 16)] = row
            return carry2

        lax.fori_loop(0, IDX_PC // 16, perm_body, 0, unroll=4)

        copies = [
            pltpu.async_copy(
                table_hbm.at[idx_v.at[pl.ds(j * SEG, SEG)]],
                rows_v.at[pl.ds(j * SEG, SEG), :],
                gsem,
            )
            for j in range(NSEG)
        ]
        for c in copies:
            c.wait()

        def row_body(b, carry2):
            bb = b * F
            va = vals_v[pl.ds(bb, 16)]
            vb = vals_v[pl.ds(bb + 16, 16)]
            s = jnp.zeros((16,), jnp.float32)
            q = jnp.zeros((16,), jnp.float32)
            for f in range(F):
                v = jnp.full((16,), va[f] if f < 16 else vb[f - 16],
                             jnp.float32)
                r = rows_v[bb + f, :]
                t = v * r
                s = s + t
                q = q + t * t
            out_v[b, :] = 0.5 * s * s - q
            return carry2

        lax.fori_loop(0, CB, row_body, 0, unroll=2)
        pltpu.sync_copy(out_v, out_hbm.at[pl.ds(base, CB)])
        return carry

    lax.fori_loop(0, NCHUNK, chunk_body, 0)


@jax.jit
def _nfm_sc(ids_flat, vals_flat, table):
    mesh = plsc.VectorSubcoreMesh(core_axis_name="c", subcore_axis_name="s")
    return pl.kernel(
        _nfm_body,
        out_type=jax.ShapeDtypeStruct((B, D), jnp.float32),
        mesh=mesh,
        compiler_params=pltpu.CompilerParams(use_tc_tiling_on_sc=False),
        scratch_types=[
            pltpu.VMEM((IDX_PC,), jnp.int32),
            pltpu.VMEM((IDX_PC + 32,), jnp.float32),
            pltpu.VMEM((IDX_PC,), jnp.int32),
            pltpu.VMEM((IDX_PC, D), jnp.float32),
            pltpu.VMEM((CB, D), jnp.float32),
            pltpu.SemaphoreType.DMA,
        ],
    )(ids_flat, vals_flat, table)


def kernel(feature_ids, feature_vals, embedding_table):
    ids_flat = feature_ids.reshape(-1).astype(jnp.int32)
    vals_flat = feature_vals.reshape(-1)
    table128 = _retile(embedding_table.T)
    table_lin = table128.reshape(-1).reshape(PROWS, D)
    return _nfm_sc(ids_flat, vals_flat, table_lin)


# speedup vs baseline: 3.3003x; 1.1561x over previous
import functools

import jax
import jax.numpy as jnp
from jax import lax
from jax.experimental import pallas as pl
from jax.experimental.pallas import tpu as pltpu
from jax.experimental.pallas import tpu_sc as plsc

B = 16384
F = 26
D = 16
NROWS = 1000000
NC, NS = 2, 16
NW = NC * NS
ROWS_PW = B // NW
CB = 64
NCHUNK = ROWS_PW // CB
IDX_PC = CB * F
SEG = 128
NSEG = IDX_PC // SEG

RBLK = 16384
RGRID = -(-NROWS // RBLK)
PROWS = RGRID * RBLK


def _retile_body(x_ref, o_ref):
    for m in range(RBLK // 1024):
        X = jnp.concatenate(
            [x_ref[:, (8 * m + k) * 128:(8 * m + k + 1) * 128]
             for k in range(8)], axis=0)
        o_ref[m * 128:(m + 1) * 128, :] = X.T


@jax.jit
def _retile(table_t):
    return pl.pallas_call(
        _retile_body,
        grid=(RGRID,),
        in_specs=[pl.BlockSpec((D, RBLK), lambda j: (0, j))],
        out_specs=pl.BlockSpec((RBLK * D // 128, 128), lambda j: (j, 0)),
        out_shape=jax.ShapeDtypeStruct((PROWS * D // 128, 128), jnp.float32),
    )(table_t)


def _nfm_body(ids_hbm, vals_hbm, table_hbm, out_hbm,
              ids_v, vals_v, idx_v, rows_v, out_v, gsem):
    wid = lax.axis_index("s") * NC + lax.axis_index("c")

    def chunk_body(ci, carry):
        base = wid * ROWS_PW + ci * CB
        fbase = base * F
        pltpu.sync_copy(ids_hbm.at[pl.ds(fbase, IDX_PC)], ids_v)
        pltpu.sync_copy(vals_hbm.at[pl.ds(fbase, IDX_PC)],
                        vals_v.at[pl.ds(0, IDX_PC)])

        def perm_body(k, carry2):
            i = ids_v[pl.ds(k * 16, 16)]
            row = ((i & ~jnp.int32(1023))
                   + ((i & jnp.int32(127)) << 3)
                   + ((i >> 7) & jnp.int32(7)))
            idx_v[pl.ds(k * 16, 16)] = row
            return carry2

        lax.fori_loop(0, IDX_PC // 16, perm_body, 0, unroll=4)

        copies = [
            pltpu.async_copy(
                table_hbm.at[idx_v.at[pl.ds(j * SEG, SEG)]],
                rows_v.at[pl.ds(j * SEG, SEG), :],
                gsem,
            )
            for j in range(NSEG)
        ]
        for c in copies:
            c.wait()

        def row_body(b, carry2):
            bb = b * F
            va = vals_v[pl.ds(bb, 16)]
            vb = vals_v[pl.ds(bb + 16, 16)]
            s = jnp.zeros((16,), jnp.float32)
            q = jnp.zeros((16,), jnp.float32)
            for f in range(F):
                v = jnp.full((16,), va[f] if f < 16 else vb[f - 16],
                             jnp.float32)
                r = rows_v[bb + f, :]
                t = v * r
                s = s + t
                q = q + t * t
            out_v[b, :] = 0.5 * s * s - q
            return carry2

        lax.fori_loop(0, CB, row_body, 0, unroll=2)
        pltpu.sync_copy(out_v, out_hbm.at[pl.ds(base, CB)])
        return carry

    lax.fori_loop(0, NCHUNK, chunk_body, 0)


@jax.jit
def _nfm_sc(ids_flat, vals_flat, table):
    mesh = plsc.VectorSubcoreMesh(core_axis_name="c", subcore_axis_name="s")
    return pl.kernel(
        _nfm_body,
        out_type=jax.ShapeDtypeStruct((B, D), jnp.float32),
        mesh=mesh,
        compiler_params=pltpu.CompilerParams(use_tc_tiling_on_sc=False),
        scratch_types=[
            pltpu.VMEM((IDX_PC,), jnp.int32),
            pltpu.VMEM((IDX_PC + 32,), jnp.float32),
            pltpu.VMEM((IDX_PC,), jnp.int32),
            pltpu.VMEM((IDX_PC, D), jnp.float32),
            pltpu.VMEM((CB, D), jnp.float32),
            pltpu.SemaphoreType.DMA,
        ],
    )(ids_flat, vals_flat, table)


def kernel(feature_ids, feature_vals, embedding_table):
    ids_flat = feature_ids.reshape(-1).astype(jnp.int32)
    vals_flat = feature_vals.reshape(-1)
    table128 = _retile(embedding_table.T)
    table_lin = table128.reshape(-1).reshape(PROWS, D)
    return _nfm_sc(ids_flat, vals_flat, table_lin)


# speedup vs baseline: 3.6631x; 1.1100x over previous
import functools

import jax
import jax.numpy as jnp
from jax import lax
from jax.experimental import pallas as pl
from jax.experimental.pallas import tpu as pltpu
from jax.experimental.pallas import tpu_sc as plsc

B = 16384
F = 26
D = 16
NROWS = 1000000
NC, NS = 2, 16
NW = NC * NS
ROWS_PW = B // NW
CB = 64
NCHUNK = ROWS_PW // CB
IDX_PC = CB * F
SEG = 128
NSEG = IDX_PC // SEG

RBLK = 32768
RGRID = -(-NROWS // RBLK)
PROWS = RGRID * RBLK


def _retile_body(x_ref, o_ref):
    for m in range(RBLK // 1024):
        X = jnp.concatenate(
            [x_ref[:, (8 * m + k) * 128:(8 * m + k + 1) * 128]
             for k in range(8)], axis=0)
        o_ref[m * 128:(m + 1) * 128, :] = X.T


@jax.jit
def _retile(table_t):
    return pl.pallas_call(
        _retile_body,
        grid=(RGRID,),
        in_specs=[pl.BlockSpec((D, RBLK), lambda j: (0, j))],
        out_specs=pl.BlockSpec((RBLK * D // 128, 128), lambda j: (j, 0)),
        out_shape=jax.ShapeDtypeStruct((PROWS * D // 128, 128), jnp.float32),
    )(table_t)


def _nfm_body(ids_hbm, vals_hbm, table_hbm, out_hbm,
              ids_v, vals_v, idx_v, rows_v, out_v, gsem):
    wid = lax.axis_index("s") * NC + lax.axis_index("c")

    def chunk_body(ci, carry):
        base = wid * ROWS_PW + ci * CB
        fbase = base * F
        pltpu.sync_copy(ids_hbm.at[pl.ds(fbase, IDX_PC)], ids_v)
        pltpu.sync_copy(vals_hbm.at[pl.ds(fbase, IDX_PC)],
                        vals_v.at[pl.ds(0, IDX_PC)])

        def perm_body(k, carry2):
            i = ids_v[pl.ds(k * 16, 16)]
            row = ((i & ~jnp.int32(1023))
                   + ((i & jnp.int32(127)) << 3)
                   + ((i >> 7) & jnp.int32(7)))
            idx_v[pl.ds(k * 16, 16)] = row
            return carry2

        lax.fori_loop(0, IDX_PC // 16, perm_body, 0, unroll=4)

        copies = [
            pltpu.async_copy(
                table_hbm.at[idx_v.at[pl.ds(j * SEG, SEG)]],
                rows_v.at[pl.ds(j * SEG, SEG), :],
                gsem,
            )
            for j in range(NSEG)
        ]
        for c in copies:
            c.wait()

        def row_body(b, carry2):
            bb = b * F
            va = vals_v[pl.ds(bb, 16)]
            vb = vals_v[pl.ds(bb + 16, 16)]
            s = jnp.zeros((16,), jnp.float32)
            q = jnp.zeros((16,), jnp.float32)
            for f in range(F):
                v = jnp.full((16,), va[f] if f < 16 else vb[f - 16],
                             jnp.float32)
                r = rows_v[bb + f, :]
                t = v * r
                s = s + t
                q = q + t * t
            out_v[b, :] = 0.5 * s * s - q
            return carry2

        lax.fori_loop(0, CB, row_body, 0, unroll=2)
        pltpu.sync_copy(out_v, out_hbm.at[pl.ds(base, CB)])
        return carry

    lax.fori_loop(0, NCHUNK, chunk_body, 0)


@jax.jit
def _nfm_sc(ids_flat, vals_flat, table):
    mesh = plsc.VectorSubcoreMesh(core_axis_name="c", subcore_axis_name="s")
    return pl.kernel(
        _nfm_body,
        out_type=jax.ShapeDtypeStruct((B, D), jnp.float32),
        mesh=mesh,
        compiler_params=pltpu.CompilerParams(use_tc_tiling_on_sc=False),
        scratch_types=[
            pltpu.VMEM((IDX_PC,), jnp.int32),
            pltpu.VMEM((IDX_PC + 32,), jnp.float32),
            pltpu.VMEM((IDX_PC,), jnp.int32),
            pltpu.VMEM((IDX_PC, D), jnp.float32),
            pltpu.VMEM((CB, D), jnp.float32),
            pltpu.SemaphoreType.DMA,
        ],
    )(ids_flat, vals_flat, table)


def kernel(feature_ids, feature_vals, embedding_table):
    ids_flat = feature_ids.reshape(-1).astype(jnp.int32)
    vals_flat = feature_vals.reshape(-1)
    table128 = _retile(embedding_table.T)
    table_lin = table128.reshape(-1).reshape(PROWS, D)
    return _nfm_sc(ids_flat, vals_flat, table_lin)


# speedup vs baseline: 3.8126x; 1.0408x over previous
import functools

import jax
import jax.numpy as jnp
from jax import lax
from jax.experimental import pallas as pl
from jax.experimental.pallas import tpu as pltpu
from jax.experimental.pallas import tpu_sc as plsc

B = 16384
F = 26
D = 16
NROWS = 1000000
NC, NS = 2, 16
NW = NC * NS
ROWS_PW = B // NW
CB = 64
NCHUNK = ROWS_PW // CB
IDX_PC = CB * F
SEG = 128
NSEG = IDX_PC // SEG

RBLK = 65536
RGRID = -(-NROWS // RBLK)
PROWS = RGRID * RBLK


def _retile_body(x_ref, o_ref):
    for m in range(RBLK // 1024):
        X = jnp.concatenate(
            [x_ref[:, (8 * m + k) * 128:(8 * m + k + 1) * 128]
             for k in range(8)], axis=0)
        o_ref[m * 128:(m + 1) * 128, :] = X.T


@jax.jit
def _retile(table_t):
    return pl.pallas_call(
        _retile_body,
        grid=(RGRID,),
        in_specs=[pl.BlockSpec((D, RBLK), lambda j: (0, j))],
        out_specs=pl.BlockSpec((RBLK * D // 128, 128), lambda j: (j, 0)),
        out_shape=jax.ShapeDtypeStruct((PROWS * D // 128, 128), jnp.float32),
    )(table_t)


def _nfm_body(ids_hbm, vals_hbm, table_hbm, out_hbm,
              ids_v, vals_v, idx_v, rows_v, out_v, gsem):
    wid = lax.axis_index("s") * NC + lax.axis_index("c")

    def chunk_body(ci, carry):
        base = wid * ROWS_PW + ci * CB
        fbase = base * F
        pltpu.sync_copy(ids_hbm.at[pl.ds(fbase, IDX_PC)], ids_v)
        pltpu.sync_copy(vals_hbm.at[pl.ds(fbase, IDX_PC)],
                        vals_v.at[pl.ds(0, IDX_PC)])

        def perm_body(k, carry2):
            i = ids_v[pl.ds(k * 16, 16)]
            row = ((i & ~jnp.int32(1023))
                   + ((i & jnp.int32(127)) << 3)
                   + ((i >> 7) & jnp.int32(7)))
            idx_v[pl.ds(k * 16, 16)] = row
            return carry2

        lax.fori_loop(0, IDX_PC // 16, perm_body, 0, unroll=4)

        copies = [
            pltpu.async_copy(
                table_hbm.at[idx_v.at[pl.ds(j * SEG, SEG)]],
                rows_v.at[pl.ds(j * SEG, SEG), :],
                gsem,
            )
            for j in range(NSEG)
        ]
        for c in copies:
            c.wait()

        def row_body(b, carry2):
            bb = b * F
            va = vals_v[pl.ds(bb, 16)]
            vb = vals_v[pl.ds(bb + 16, 16)]
            s = jnp.zeros((16,), jnp.float32)
            q = jnp.zeros((16,), jnp.float32)
            for f in range(F):
                v = jnp.full((16,), va[f] if f < 16 else vb[f - 16],
                             jnp.float32)
                r = rows_v[bb + f, :]
                t = v * r
                s = s + t
                q = q + t * t
            out_v[b, :] = 0.5 * s * s - q
            return carry2

        lax.fori_loop(0, CB, row_body, 0, unroll=2)
        pltpu.sync_copy(out_v, out_hbm.at[pl.ds(base, CB)])
        return carry

    lax.fori_loop(0, NCHUNK, chunk_body, 0)


@jax.jit
def _nfm_sc(ids_flat, vals_flat, table):
    mesh = plsc.VectorSubcoreMesh(core_axis_name="c", subcore_axis_name="s")
    return pl.kernel(
        _nfm_body,
        out_type=jax.ShapeDtypeStruct((B, D), jnp.float32),
        mesh=mesh,
        compiler_params=pltpu.CompilerParams(use_tc_tiling_on_sc=False),
        scratch_types=[
            pltpu.VMEM((IDX_PC,), jnp.int32),
            pltpu.VMEM((IDX_PC + 32,), jnp.float32),
            pltpu.VMEM((IDX_PC,), jnp.int32),
            pltpu.VMEM((IDX_PC, D), jnp.float32),
            pltpu.VMEM((CB, D), jnp.float32),
            pltpu.SemaphoreType.DMA,
        ],
    )(ids_flat, vals_flat, table)


def kernel(feature_ids, feature_vals, embedding_table):
    ids_flat = feature_ids.reshape(-1).astype(jnp.int32)
    vals_flat = feature_vals.reshape(-1)
    table128 = _retile(embedding_table.T)
    table_lin = table128.reshape(-1).reshape(PROWS, D)
    return _nfm_sc(ids_flat, vals_flat, table_lin)


# speedup vs baseline: 3.8450x; 1.0085x over previous
import functools

import jax
import jax.numpy as jnp
from jax import lax
from jax.experimental import pallas as pl
from jax.experimental.pallas import tpu as pltpu
from jax.experimental.pallas import tpu_sc as plsc

B = 16384
F = 26
D = 16
NROWS = 1000000
NC, NS = 2, 16
NW = NC * NS
ROWS_PW = B // NW
CB = 64
NCHUNK = ROWS_PW // CB
IDX_PC = CB * F
SEG = 128
NSEG = IDX_PC // SEG

RBLK = 131072
RGRID = -(-NROWS // RBLK)
PROWS = RGRID * RBLK


def _retile_body(x_ref, o_ref):
    for m in range(RBLK // 1024):
        X = jnp.concatenate(
            [x_ref[:, (8 * m + k) * 128:(8 * m + k + 1) * 128]
             for k in range(8)], axis=0)
        o_ref[m * 128:(m + 1) * 128, :] = X.T


@jax.jit
def _retile(table_t):
    return pl.pallas_call(
        _retile_body,
        grid=(RGRID,),
        in_specs=[pl.BlockSpec((D, RBLK), lambda j: (0, j))],
        out_specs=pl.BlockSpec((RBLK * D // 128, 128), lambda j: (j, 0)),
        out_shape=jax.ShapeDtypeStruct((PROWS * D // 128, 128), jnp.float32),
    )(table_t)


def _nfm_body(ids_hbm, vals_hbm, table_hbm, out_hbm,
              ids_v, vals_v, idx_v, rows_v, out_v, gsem):
    wid = lax.axis_index("s") * NC + lax.axis_index("c")

    def chunk_body(ci, carry):
        base = wid * ROWS_PW + ci * CB
        fbase = base * F
        pltpu.sync_copy(ids_hbm.at[pl.ds(fbase, IDX_PC)], ids_v)
        pltpu.sync_copy(vals_hbm.at[pl.ds(fbase, IDX_PC)],
                        vals_v.at[pl.ds(0, IDX_PC)])

        def perm_body(k, carry2):
            i = ids_v[pl.ds(k * 16, 16)]
            row = ((i & ~jnp.int32(1023))
                   + ((i & jnp.int32(127)) << 3)
                   + ((i >> 7) & jnp.int32(7)))
            idx_v[pl.ds(k * 16, 16)] = row
            return carry2

        lax.fori_loop(0, IDX_PC // 16, perm_body, 0, unroll=4)

        copies = [
            pltpu.async_copy(
                table_hbm.at[idx_v.at[pl.ds(j * SEG, SEG)]],
                rows_v.at[pl.ds(j * SEG, SEG), :],
                gsem,
            )
            for j in range(NSEG)
        ]
        for c in copies:
            c.wait()

        def row_body(b, carry2):
            bb = b * F
            va = vals_v[pl.ds(bb, 16)]
            vb = vals_v[pl.ds(bb + 16, 16)]
            s = jnp.zeros((16,), jnp.float32)
            q = jnp.zeros((16,), jnp.float32)
            for f in range(F):
                v = jnp.full((16,), va[f] if f < 16 else vb[f - 16],
                             jnp.float32)
                r = rows_v[bb + f, :]
                t = v * r
                s = s + t
                q = q + t * t
            out_v[b, :] = 0.5 * s * s - q
            return carry2

        lax.fori_loop(0, CB, row_body, 0, unroll=2)
        pltpu.sync_copy(out_v, out_hbm.at[pl.ds(base, CB)])
        return carry

    lax.fori_loop(0, NCHUNK, chunk_body, 0)


@jax.jit
def _nfm_sc(ids_flat, vals_flat, table):
    mesh = plsc.VectorSubcoreMesh(core_axis_name="c", subcore_axis_name="s")
    return pl.kernel(
        _nfm_body,
        out_type=jax.ShapeDtypeStruct((B, D), jnp.float32),
        mesh=mesh,
        compiler_params=pltpu.CompilerParams(use_tc_tiling_on_sc=False),
        scratch_types=[
            pltpu.VMEM((IDX_PC,), jnp.int32),
            pltpu.VMEM((IDX_PC + 32,), jnp.float32),
            pltpu.VMEM((IDX_PC,), jnp.int32),
            pltpu.VMEM((IDX_PC, D), jnp.float32),
            pltpu.VMEM((CB, D), jnp.float32),
            pltpu.SemaphoreType.DMA,
        ],
    )(ids_flat, vals_flat, table)


def kernel(feature_ids, feature_vals, embedding_table):
    ids_flat = feature_ids.reshape(-1).astype(jnp.int32)
    vals_flat = feature_vals.reshape(-1)
    table128 = _retile(embedding_table.T)
    table_lin = table128.reshape(-1).reshape(PROWS, D)
    return _nfm_sc(ids_flat, vals_flat, table_lin)


# speedup vs baseline: 5.2366x; 1.3619x over previous
import functools

import jax
import jax.numpy as jnp
from jax import lax
from jax.experimental import pallas as pl
from jax.experimental.pallas import tpu as pltpu
from jax.experimental.pallas import tpu_sc as plsc

B = 16384
F = 26
D = 16
NROWS = 1000000
NC, NS = 2, 16
NW = NC * NS
ROWS_PW = B // NW
CB = 64
NCHUNK = ROWS_PW // CB
IDX_PC = CB * F
SEG = 128
NSEG = IDX_PC // SEG

RBLK = 131072
RGRID = -(-NROWS // RBLK)
PROWS = RGRID * RBLK


def _retile_body(x_ref, o_ref):
    for m in range(RBLK // 1024):
        X = jnp.concatenate(
            [x_ref[:, (8 * m + k) * 128:(8 * m + k + 1) * 128]
             for k in range(8)], axis=0)
        o_ref[m * 128:(m + 1) * 128, :] = X.T


@jax.jit
def _retile(table_t):
    return pl.pallas_call(
        _retile_body,
        grid=(RGRID,),
        in_specs=[pl.BlockSpec((D, RBLK), lambda j: (0, j))],
        out_specs=pl.BlockSpec((RBLK * D // 128, 128), lambda j: (j, 0)),
        out_shape=jax.ShapeDtypeStruct((PROWS * D // 128, 128), jnp.float32),
    )(table_t)


def _nfm_body(ids_hbm, vals_hbm, table_hbm, out_hbm,
              ids_v, vals_v, idx_v, rows_v, out_v, gsem):
    wid = lax.axis_index("s") * NC + lax.axis_index("c")

    def stage(ci, buf):
        base = wid * ROWS_PW + ci * CB
        pltpu.sync_copy(ids_hbm.at[:, pl.ds(base, CB)], ids_v[buf])
        pltpu.sync_copy(vals_hbm.at[:, pl.ds(base, CB)], vals_v[buf])

        def perm_body(f, carry):
            for g in range(CB // 16):
                i = ids_v[buf][f, pl.ds(g * 16, 16)]
                row = ((i & ~jnp.int32(1023))
                       + ((i & jnp.int32(127)) << 3)
                       + ((i >> 7) & jnp.int32(7)))
                idx_v[buf][pl.ds(f * CB + g * 16, 16)] = row
            return carry

        lax.fori_loop(0, F, perm_body, 0)
        return [
            pltpu.async_copy(
                table_hbm.at[idx_v[buf].at[pl.ds(j * SEG, SEG)]],
                rows_v[buf].at[pl.ds(j * SEG, SEG), :],
                gsem[buf],
            )
            for j in range(NSEG)
        ]

    def wait_gathers(buf):
        for j in range(NSEG):
            pltpu.make_async_copy(
                table_hbm.at[idx_v[buf].at[pl.ds(j * SEG, SEG)]],
                rows_v[buf].at[pl.ds(j * SEG, SEG), :],
                gsem[buf],
            ).wait()

    def compute(ci, buf):
        base = wid * ROWS_PW + ci * CB
        wait_gathers(buf)

        def group(g, carry):
            vv = [vals_v[buf][f, pl.ds(g * 16, 16)] for f in range(F)]
            for b in range(16):
                gb = g * 16 + b
                s = jnp.zeros((16,), jnp.float32)
                q = jnp.zeros((16,), jnp.float32)
                for f in range(F):
                    v = jnp.full((16,), vv[f][b], jnp.float32)
                    r = rows_v[buf][f * CB + gb, :]
                    t = v * r
                    s = s + t
                    q = q + t * t
                out_v[gb, :] = 0.5 * s * s - q
            return carry

        lax.fori_loop(0, CB // 16, group, 0)
        pltpu.sync_copy(out_v, out_hbm.at[pl.ds(base, CB)])

    stage(0, 0)

    def pipe(ci2, carry):
        ci = ci2 * 2
        stage(ci + 1, 1)
        compute(ci, 0)

        @pl.when(ci + 2 < NCHUNK)
        def _():
            stage(ci + 2, 0)

        compute(ci + 1, 1)
        return carry

    lax.fori_loop(0, NCHUNK // 2, pipe, 0)


@jax.jit
def _nfm_sc(ids_t, vals_t, table):
    mesh = plsc.VectorSubcoreMesh(core_axis_name="c", subcore_axis_name="s")
    return pl.kernel(
        _nfm_body,
        out_type=jax.ShapeDtypeStruct((B, D), jnp.float32),
        mesh=mesh,
        compiler_params=pltpu.CompilerParams(use_tc_tiling_on_sc=False),
        scratch_types=[
            [pltpu.VMEM((F, CB), jnp.int32) for _ in range(2)],
            [pltpu.VMEM((F, CB), jnp.float32) for _ in range(2)],
            [pltpu.VMEM((IDX_PC,), jnp.int32) for _ in range(2)],
            [pltpu.VMEM((IDX_PC, D), jnp.float32) for _ in range(2)],
            pltpu.VMEM((CB, D), jnp.float32),
            [pltpu.SemaphoreType.DMA for _ in range(2)],
        ],
    )(ids_t, vals_t, table)


def kernel(feature_ids, feature_vals, embedding_table):
    ids_t = feature_ids.T.astype(jnp.int32)
    vals_t = feature_vals.T
    table128 = _retile(embedding_table.T)
    table_lin = table128.reshape(-1).reshape(PROWS, D)
    return _nfm_sc(ids_t, vals_t, table_lin)


# speedup vs baseline: 5.3443x; 1.0206x over previous
import functools

import jax
import jax.numpy as jnp
from jax import lax
from jax.experimental import pallas as pl
from jax.experimental.pallas import tpu as pltpu
from jax.experimental.pallas import tpu_sc as plsc

B = 16384
F = 26
D = 16
NROWS = 1000000
NC, NS = 2, 16
NW = NC * NS
ROWS_PW = B // NW
CB = 64
NCHUNK = ROWS_PW // CB
IDX_PC = CB * F
SEG = 128
NSEG = IDX_PC // SEG

RBLK = 131072
RGRID = -(-NROWS // RBLK)
PROWS = RGRID * RBLK


def _retile_body(x_ref, o_ref):
    for m in range(RBLK // 1024):
        X = jnp.concatenate(
            [x_ref[:, (8 * m + k) * 128:(8 * m + k + 1) * 128]
             for k in range(8)], axis=0)
        o_ref[m * 128:(m + 1) * 128, :] = X.T


@jax.jit
def _retile(table_t):
    return pl.pallas_call(
        _retile_body,
        grid=(RGRID,),
        in_specs=[pl.BlockSpec((D, RBLK), lambda j: (0, j))],
        out_specs=pl.BlockSpec((RBLK * D // 128, 128), lambda j: (j, 0)),
        out_shape=jax.ShapeDtypeStruct((PROWS * D // 128, 128), jnp.float32),
    )(table_t)


def _nfm_body(ids_hbm, vals_hbm, table_hbm, out_hbm,
              ids_v, vals_v, idx_v, rows_v, out_v, gsem):
    wid = lax.axis_index("s") * NC + lax.axis_index("c")

    def stage(ci, buf):
        base = wid * ROWS_PW + ci * CB
        pltpu.sync_copy(ids_hbm.at[:, pl.ds(base, CB)], ids_v[buf])
        pltpu.sync_copy(vals_hbm.at[:, pl.ds(base, CB)], vals_v[buf])

        def perm_body(f, carry):
            for g in range(CB // 16):
                i = ids_v[buf][f, pl.ds(g * 16, 16)]
                row = ((i & ~jnp.int32(1023))
                       + ((i & jnp.int32(127)) << 3)
                       + ((i >> 7) & jnp.int32(7)))
                idx_v[buf][pl.ds(f * CB + g * 16, 16)] = row
            return carry

        lax.fori_loop(0, F, perm_body, 0)
        return [
            pltpu.async_copy(
                table_hbm.at[idx_v[buf].at[pl.ds(j * SEG, SEG)]],
                rows_v[buf].at[pl.ds(j * SEG, SEG), :],
                gsem[buf],
            )
            for j in range(NSEG)
        ]

    def wait_gathers(buf):
        for j in range(NSEG):
            pltpu.make_async_copy(
                table_hbm.at[idx_v[buf].at[pl.ds(j * SEG, SEG)]],
                rows_v[buf].at[pl.ds(j * SEG, SEG), :],
                gsem[buf],
            ).wait()

    def compute(ci, buf):
        base = wid * ROWS_PW + ci * CB
        wait_gathers(buf)

        def group(g, carry):
            vv = [vals_v[buf][f, pl.ds(g * 16, 16)] for f in range(F)]
            for b in range(16):
                gb = g * 16 + b
                sa = [jnp.zeros((16,), jnp.float32) for _ in range(4)]
                qa = [jnp.zeros((16,), jnp.float32) for _ in range(4)]
                for f in range(F):
                    v = jnp.full((16,), vv[f][b], jnp.float32)
                    r = rows_v[buf][f * CB + gb, :]
                    t = v * r
                    sa[f % 4] = sa[f % 4] + t
                    qa[f % 4] = qa[f % 4] + t * t
                s = (sa[0] + sa[1]) + (sa[2] + sa[3])
                q = (qa[0] + qa[1]) + (qa[2] + qa[3])
                out_v[gb, :] = 0.5 * s * s - q
            return carry

        lax.fori_loop(0, CB // 16, group, 0)
        pltpu.sync_copy(out_v, out_hbm.at[pl.ds(base, CB)])

    stage(0, 0)

    def pipe(ci2, carry):
        ci = ci2 * 2
        stage(ci + 1, 1)
        compute(ci, 0)

        @pl.when(ci + 2 < NCHUNK)
        def _():
            stage(ci + 2, 0)

        compute(ci + 1, 1)
        return carry

    lax.fori_loop(0, NCHUNK // 2, pipe, 0)


@jax.jit
def _nfm_sc(ids_t, vals_t, table):
    mesh = plsc.VectorSubcoreMesh(core_axis_name="c", subcore_axis_name="s")
    return pl.kernel(
        _nfm_body,
        out_type=jax.ShapeDtypeStruct((B, D), jnp.float32),
        mesh=mesh,
        compiler_params=pltpu.CompilerParams(use_tc_tiling_on_sc=False),
        scratch_types=[
            [pltpu.VMEM((F, CB), jnp.int32) for _ in range(2)],
            [pltpu.VMEM((F, CB), jnp.float32) for _ in range(2)],
            [pltpu.VMEM((IDX_PC,), jnp.int32) for _ in range(2)],
            [pltpu.VMEM((IDX_PC, D), jnp.float32) for _ in range(2)],
            pltpu.VMEM((CB, D), jnp.float32),
            [pltpu.SemaphoreType.DMA for _ in range(2)],
        ],
    )(ids_t, vals_t, table)


def kernel(feature_ids, feature_vals, embedding_table):
    ids_t = feature_ids.T.astype(jnp.int32)
    vals_t = feature_vals.T
    table128 = _retile(embedding_table.T)
    table_lin = table128.reshape(-1).reshape(PROWS, D)
    return _nfm_sc(ids_t, vals_t, table_lin)


# speedup vs baseline: 5.5558x; 1.0396x over previous
import functools

import jax
import jax.numpy as jnp
from jax import lax
from jax.experimental import pallas as pl
from jax.experimental.pallas import tpu as pltpu
from jax.experimental.pallas import tpu_sc as plsc

B = 16384
F = 26
D = 16
NROWS = 1000000
NC, NS = 2, 16
NW = NC * NS
ROWS_PW = B // NW
CB = 128
NCHUNK = ROWS_PW // CB
IDX_PC = CB * F
SEG = 128
NSEG = IDX_PC // SEG

RBLK = 131072
RGRID = -(-NROWS // RBLK)
PROWS = RGRID * RBLK


def _retile_body(x_ref, o_ref):
    for m in range(RBLK // 1024):
        X = jnp.concatenate(
            [x_ref[:, (8 * m + k) * 128:(8 * m + k + 1) * 128]
             for k in range(8)], axis=0)
        o_ref[m * 128:(m + 1) * 128, :] = X.T


@jax.jit
def _retile(table_t):
    return pl.pallas_call(
        _retile_body,
        grid=(RGRID,),
        in_specs=[pl.BlockSpec((D, RBLK), lambda j: (0, j))],
        out_specs=pl.BlockSpec((RBLK * D // 128, 128), lambda j: (j, 0)),
        out_shape=jax.ShapeDtypeStruct((PROWS * D // 128, 128), jnp.float32),
    )(table_t)


def _nfm_body(ids_hbm, vals_hbm, table_hbm, out_hbm,
              ids_v, vals_v, idx_v, rows_v, out_v, gsem):
    wid = lax.axis_index("s") * NC + lax.axis_index("c")

    def stage(ci, buf):
        base = wid * ROWS_PW + ci * CB
        pltpu.sync_copy(ids_hbm.at[:, pl.ds(base, CB)], ids_v[buf])
        pltpu.sync_copy(vals_hbm.at[:, pl.ds(base, CB)], vals_v[buf])

        def perm_body(f, carry):
            for g in range(CB // 16):
                i = ids_v[buf][f, pl.ds(g * 16, 16)]
                row = ((i & ~jnp.int32(1023))
                       + ((i & jnp.int32(127)) << 3)
                       + ((i >> 7) & jnp.int32(7)))
                idx_v[buf][pl.ds(f * CB + g * 16, 16)] = row
            return carry

        lax.fori_loop(0, F, perm_body, 0)
        return [
            pltpu.async_copy(
                table_hbm.at[idx_v[buf].at[pl.ds(j * SEG, SEG)]],
                rows_v[buf].at[pl.ds(j * SEG, SEG), :],
                gsem[buf],
            )
            for j in range(NSEG)
        ]

    def wait_gathers(buf):
        for j in range(NSEG):
            pltpu.make_async_copy(
                table_hbm.at[idx_v[buf].at[pl.ds(j * SEG, SEG)]],
                rows_v[buf].at[pl.ds(j * SEG, SEG), :],
                gsem[buf],
            ).wait()

    def compute(ci, buf):
        base = wid * ROWS_PW + ci * CB
        wait_gathers(buf)

        def group(g, carry):
            vv = [vals_v[buf][f, pl.ds(g * 16, 16)] for f in range(F)]
            for b in range(16):
                gb = g * 16 + b
                sa = [jnp.zeros((16,), jnp.float32) for _ in range(4)]
                qa = [jnp.zeros((16,), jnp.float32) for _ in range(4)]
                for f in range(F):
                    v = jnp.full((16,), vv[f][b], jnp.float32)
                    r = rows_v[buf][f * CB + gb, :]
                    t = v * r
                    sa[f % 4] = sa[f % 4] + t
                    qa[f % 4] = qa[f % 4] + t * t
                s = (sa[0] + sa[1]) + (sa[2] + sa[3])
                q = (qa[0] + qa[1]) + (qa[2] + qa[3])
                out_v[gb, :] = 0.5 * s * s - q
            return carry

        lax.fori_loop(0, CB // 16, group, 0)
        pltpu.sync_copy(out_v, out_hbm.at[pl.ds(base, CB)])

    stage(0, 0)

    def pipe(ci2, carry):
        ci = ci2 * 2
        stage(ci + 1, 1)
        compute(ci, 0)

        @pl.when(ci + 2 < NCHUNK)
        def _():
            stage(ci + 2, 0)

        compute(ci + 1, 1)
        return carry

    lax.fori_loop(0, NCHUNK // 2, pipe, 0)


@jax.jit
def _nfm_sc(ids_t, vals_t, table):
    mesh = plsc.VectorSubcoreMesh(core_axis_name="c", subcore_axis_name="s")
    return pl.kernel(
        _nfm_body,
        out_type=jax.ShapeDtypeStruct((B, D), jnp.float32),
        mesh=mesh,
        compiler_params=pltpu.CompilerParams(use_tc_tiling_on_sc=False),
        scratch_types=[
            [pltpu.VMEM((F, CB), jnp.int32) for _ in range(2)],
            [pltpu.VMEM((F, CB), jnp.float32) for _ in range(2)],
            [pltpu.VMEM((IDX_PC,), jnp.int32) for _ in range(2)],
            [pltpu.VMEM((IDX_PC, D), jnp.float32) for _ in range(2)],
            pltpu.VMEM((CB, D), jnp.float32),
            [pltpu.SemaphoreType.DMA for _ in range(2)],
        ],
    )(ids_t, vals_t, table)


def kernel(feature_ids, feature_vals, embedding_table):
    ids_t = feature_ids.T.astype(jnp.int32)
    vals_t = feature_vals.T
    table128 = _retile(embedding_table.T)
    table_lin = table128.reshape(-1).reshape(PROWS, D)
    return _nfm_sc(ids_t, vals_t, table_lin)
